# baseline (speedup 1.0000x reference)
import functools
import jax
import jax.numpy as jnp
from jax import lax
from jax.experimental import pallas as pl
from jax.experimental.pallas import tpu as pltpu

B, S, H, I, K = 4, 2048, 768, 3072, 384
TB = 256


def _moc_body(x_ref, gw_ref, uw_ref, dw_ref, o_ref):
    xb = x_ref[...]
    gate = lax.dot_general(xb, gw_ref[...],
                           (((1,), (1,)), ((), ())),
                           preferred_element_type=jnp.float32)
    up = lax.dot_general(xb.astype(jnp.bfloat16), uw_ref[...],
                         (((1,), (1,)), ((), ())),
                         preferred_element_type=jnp.float32)

    bits = lax.bitcast_convert_type(gate, jnp.uint32)
    ukey = jnp.where(bits >> 31 == 1, ~bits, bits | jnp.uint32(0x80000000))

    def step(i, p):
        bit = 31 - i
        cand = p | (jnp.uint32(1) << bit.astype(jnp.uint32))
        cnt = jnp.sum((ukey >= cand).astype(jnp.int32), axis=1, keepdims=True)
        return jnp.where(cnt >= K, cand, p)

    p0 = jnp.zeros((TB, 1), dtype=jnp.uint32)
    thr = lax.fori_loop(0, 32, step, p0)

    mask = ukey >= thr
    act = gate * jax.nn.sigmoid(gate) * up
    masked = jnp.where(mask, act, 0.0).astype(jnp.bfloat16)
    o_ref[...] = lax.dot_general(masked, dw_ref[...],
                                 (((1,), (1,)), ((), ())),
                                 preferred_element_type=jnp.float32)


@jax.jit
def kernel(x, gate_w, up_w, down_w):
    b, s, h = x.shape
    T = b * s
    x2 = x.reshape(T, h)
    up_w = up_w.astype(jnp.bfloat16)
    down_w = down_w.astype(jnp.bfloat16)
    out = pl.pallas_call(
        _moc_body,
        grid=(T // TB,),
        in_specs=[
            pl.BlockSpec((TB, H), lambda i: (i, 0)),
            pl.BlockSpec((I, H), lambda i: (0, 0)),
            pl.BlockSpec((I, H), lambda i: (0, 0)),
            pl.BlockSpec((H, I), lambda i: (0, 0)),
        ],
        out_specs=pl.BlockSpec((TB, H), lambda i: (i, 0)),
        out_shape=jax.ShapeDtypeStruct((T, H), jnp.float32),
    )(x2, gate_w, up_w, down_w)
    return out.reshape(b, s, h)


# speedup vs baseline: 1.0265x; 1.0265x over previous
import functools
import jax
import jax.numpy as jnp
from jax import lax
from jax.experimental import pallas as pl
from jax.experimental.pallas import tpu as pltpu

B, S, H, I, K = 4, 2048, 768, 3072, 384
TB = 256


def _moc_body(x_ref, gw_ref, uw_ref, dw_ref, o_ref):
    xb = x_ref[...]
    gate = lax.dot_general(gw_ref[...], xb,
                           (((1,), (1,)), ((), ())),
                           preferred_element_type=jnp.float32)
    up = lax.dot_general(uw_ref[...], xb.astype(jnp.bfloat16),
                         (((1,), (1,)), ((), ())),
                         preferred_element_type=jnp.float32)

    bits = lax.bitcast_convert_type(gate, jnp.uint32)
    ukey = jnp.where(bits >> 31 == 1, ~bits, bits | jnp.uint32(0x80000000))

    def step(i, p):
        bit = 31 - i
        cand = p | (jnp.uint32(1) << bit.astype(jnp.uint32))
        cnt = jnp.sum((ukey >= cand).astype(jnp.int32), axis=0, keepdims=True)
        return jnp.where(cnt >= K, cand, p)

    p0 = jnp.zeros((1, TB), dtype=jnp.uint32)
    thr = lax.fori_loop(0, 32, step, p0)

    mask = ukey >= thr
    act = gate * jax.nn.sigmoid(gate) * up
    masked = jnp.where(mask, act, 0.0).astype(jnp.bfloat16)
    o_ref[...] = lax.dot_general(masked, dw_ref[...],
                                 (((0,), (1,)), ((), ())),
                                 preferred_element_type=jnp.float32)


@jax.jit
def kernel(x, gate_w, up_w, down_w):
    b, s, h = x.shape
    T = b * s
    x2 = x.reshape(T, h)
    up_w = up_w.astype(jnp.bfloat16)
    down_w = down_w.astype(jnp.bfloat16)
    out = pl.pallas_call(
        _moc_body,
        grid=(T // TB,),
        in_specs=[
            pl.BlockSpec((TB, H), lambda i: (i, 0)),
            pl.BlockSpec((I, H), lambda i: (0, 0)),
            pl.BlockSpec((I, H), lambda i: (0, 0)),
            pl.BlockSpec((H, I), lambda i: (0, 0)),
        ],
        out_specs=pl.BlockSpec((TB, H), lambda i: (i, 0)),
        out_shape=jax.ShapeDtypeStruct((T, H), jnp.float32),
    )(x2, gate_w, up_w, down_w)
    return out.reshape(b, s, h)
